# baseline (speedup 1.0000x reference)
import dataclasses
import functools

import jax
import jax.numpy as jnp
from jax import lax
from jax.experimental import pallas as pl
from jax.experimental.pallas import tpu as pltpu
from jax.experimental.pallas import tpu_sc as plsc

B = 16384
D = 128
TAGS = 20
N_SMALL = 50
H = 256

NC = 2
NS = 16
NW = NC * NS
B_PER_W = B // NW
ROW_CHUNK = B_PER_W // 4
N_GRP = B_PER_W // 16

_sc_mesh = plsc.VectorSubcoreMesh(core_axis_name="c", subcore_axis_name="s")

_sc_cp = pltpu.CompilerParams()
if "needs_layout_passes" in pltpu.CompilerParams.__dataclass_fields__:
    _sc_cp = dataclasses.replace(_sc_cp, needs_layout_passes=False)


@functools.partial(
    pl.kernel,
    mesh=_sc_mesh,
    compiler_params=_sc_cp,
    out_type=(
        jax.ShapeDtypeStruct((B, D), jnp.float32),
        jax.ShapeDtypeStruct((B, D), jnp.float32),
    ),
    scratch_types=[
        pltpu.VMEM((B_PER_W,), jnp.int32),
        pltpu.VMEM((B_PER_W * TAGS,), jnp.int32),
        pltpu.VMEM((B_PER_W * TAGS,), jnp.int32),
        pltpu.VMEM((ROW_CHUNK, D), jnp.float32),
        pltpu.VMEM((ROW_CHUNK, D), jnp.float32),
        pltpu.VMEM((B_PER_W, D), jnp.float32),
        pltpu.SemaphoreType.DMA,
        pltpu.SemaphoreType.DMA,
        pltpu.SemaphoreType.DMA,
        pltpu.SemaphoreType.DMA,
        pltpu.SemaphoreType.DMA,
    ],
)
def _sc_gather_hist(table_hbm, idx_hbm, mood_hbm, instr_hbm,
                    rows_hbm, counts_hbm,
                    idx_v, mood_v, instr_v, rows_a, rows_b, counts_v,
                    msem, isem, gsem_a, gsem_b, csem):
    wid = lax.axis_index("s") * NC + lax.axis_index("c")
    base = wid * B_PER_W

    mood_dma = pltpu.async_copy(
        mood_hbm.at[pl.ds(base * TAGS, B_PER_W * TAGS)], mood_v, msem)
    instr_dma = pltpu.async_copy(
        instr_hbm.at[pl.ds(base * TAGS, B_PER_W * TAGS)], instr_v, isem)
    pltpu.sync_copy(idx_hbm.at[pl.ds(base, B_PER_W)], idx_v)
    g0 = pltpu.async_copy(
        table_hbm.at[idx_v.at[pl.ds(0, ROW_CHUNK)]], rows_a, gsem_a)
    g1 = pltpu.async_copy(
        table_hbm.at[idx_v.at[pl.ds(ROW_CHUNK, ROW_CHUNK)]], rows_b, gsem_b)

    zeros16 = jnp.zeros((16,), jnp.float32)

    @pl.loop(0, B_PER_W)
    def _(r):
        for u in range(D // 16):
            counts_v[r, pl.ds(u * 16, 16)] = zeros16

    mood_dma.wait()
    instr_dma.wait()
    ones16 = jnp.ones((16,), jnp.float32)
    iota16 = lax.iota(jnp.int32, 16)
    iota_tags = iota16 * TAGS

    @pl.loop(0, N_GRP)
    def _(g):
        rows = g * 16 + iota16
        flat_base = g * (16 * TAGS) + iota_tags
        for t in range(TAGS):
            mids = plsc.load_gather(mood_v, [flat_base + t])
            plsc.addupdate_scatter(counts_v, [rows, mids], ones16)
            iids = plsc.load_gather(instr_v, [flat_base + t])
            plsc.addupdate_scatter(counts_v, [rows, iids + (D // 2)], ones16)

    counts_out = pltpu.async_copy(
        counts_v, counts_hbm.at[pl.ds(base, B_PER_W)], csem)

    g0.wait()
    pltpu.sync_copy(rows_a, rows_hbm.at[pl.ds(base, ROW_CHUNK)])
    g2 = pltpu.async_copy(
        table_hbm.at[idx_v.at[pl.ds(2 * ROW_CHUNK, ROW_CHUNK)]], rows_a,
        gsem_a)
    g1.wait()
    pltpu.sync_copy(rows_b, rows_hbm.at[pl.ds(base + ROW_CHUNK, ROW_CHUNK)])
    g3 = pltpu.async_copy(
        table_hbm.at[idx_v.at[pl.ds(3 * ROW_CHUNK, ROW_CHUNK)]], rows_b,
        gsem_b)
    g2.wait()
    pltpu.sync_copy(rows_a, rows_hbm.at[pl.ds(base + 2 * ROW_CHUNK,
                                              ROW_CHUNK)])
    g3.wait()
    pltpu.sync_copy(rows_b, rows_hbm.at[pl.ds(base + 3 * ROW_CHUNK,
                                              ROW_CHUNK)])
    counts_out.wait()


BLK = 2048


def _mlp_body(genre_ref, counts_ref, tempo_ref, mt_ref, it_ref,
              wt_ref, bt_ref, w1_ref, b1_ref, w2_ref, b2_ref, out_ref):
    tempo = tempo_ref[...]
    tempo_vec = (tempo[:, 0:1] * wt_ref[0:1, :]
                 + tempo[:, 1:2] * wt_ref[1:2, :] + bt_ref[...])

    counts = counts_ref[...]
    mood_vec = jnp.dot(counts[:, :D // 2], mt_ref[...],
                       preferred_element_type=jnp.float32) * (1.0 / TAGS)
    instr_vec = jnp.dot(counts[:, D // 2:], it_ref[...],
                        preferred_element_type=jnp.float32) * (1.0 / TAGS)

    x = jnp.concatenate(
        [genre_ref[...], tempo_vec, mood_vec, instr_vec],
        axis=-1).astype(jnp.bfloat16)
    h = jnp.maximum(
        jnp.dot(x, w1_ref[...], preferred_element_type=jnp.float32)
        + b1_ref[...], 0.0).astype(jnp.bfloat16)
    out_ref[...] = (jnp.dot(h, w2_ref[...], preferred_element_type=jnp.float32)
                    + b2_ref[...])


def _tc_mlp(genre_vec, counts, tempo_range, mood_pad, instr_pad,
            Wt, bt2, W1, b12, W2, b22):
    n_blk = B // BLK
    full = lambda shape: pl.BlockSpec(shape, lambda i: (0, 0))
    return pl.pallas_call(
        _mlp_body,
        grid=(n_blk,),
        in_specs=[
            pl.BlockSpec((BLK, D), lambda i: (i, 0)),
            pl.BlockSpec((BLK, D), lambda i: (i, 0)),
            pl.BlockSpec((BLK, 2), lambda i: (i, 0)),
            full((D // 2, D)),
            full((D // 2, D)),
            full((2, D)),
            full((1, D)),
            full((4 * D, H)),
            full((1, H)),
            full((H, D)),
            full((1, D)),
        ],
        out_specs=pl.BlockSpec((BLK, D), lambda i: (i, 0)),
        out_shape=jax.ShapeDtypeStruct((B, D), jnp.float32),
        compiler_params=pltpu.CompilerParams(
            dimension_semantics=("parallel",)),
    )(genre_vec, counts, tempo_range, mood_pad, instr_pad,
      Wt, bt2, W1, b12, W2, b22)


def kernel(genre_ids, tempo_range, mood_ids, instr_ids, genre_table,
           mood_table, instr_table, Wt, bt, W1, b1, W2, b2):
    genre_vec, counts = _sc_gather_hist(
        genre_table, genre_ids.astype(jnp.int32),
        mood_ids.astype(jnp.int32).reshape(B * TAGS),
        instr_ids.astype(jnp.int32).reshape(B * TAGS))

    pad = jnp.zeros((D // 2 - N_SMALL, D), jnp.float32)
    mood_pad = jnp.concatenate([mood_table, pad], axis=0)
    instr_pad = jnp.concatenate([instr_table, pad], axis=0)

    return _tc_mlp(genre_vec, counts, tempo_range, mood_pad, instr_pad,
                   Wt, bt.reshape(1, D), W1.astype(jnp.bfloat16),
                   b1.reshape(1, H), W2.astype(jnp.bfloat16),
                   b2.reshape(1, D))


# speedup vs baseline: 1.4494x; 1.4494x over previous
import dataclasses
import functools

import jax
import jax.numpy as jnp
from jax import lax
from jax.experimental import pallas as pl
from jax.experimental.pallas import tpu as pltpu
from jax.experimental.pallas import tpu_sc as plsc

B = 16384
D = 128
TAGS = 20
N_SMALL = 50
H = 256

NC = 2
NS = 16
NW = NC * NS
B_PER_W = B // NW
ROW_CHUNK = B_PER_W // 4
N_GRP = B_PER_W // 16

_sc_mesh = plsc.VectorSubcoreMesh(core_axis_name="c", subcore_axis_name="s")

_sc_cp = pltpu.CompilerParams()
if "needs_layout_passes" in pltpu.CompilerParams.__dataclass_fields__:
    _sc_cp = dataclasses.replace(_sc_cp, needs_layout_passes=False)


@functools.partial(
    pl.kernel,
    mesh=_sc_mesh,
    compiler_params=_sc_cp,
    out_type=(
        jax.ShapeDtypeStruct((B, D), jnp.float32),
        jax.ShapeDtypeStruct((B, D), jnp.float32),
    ),
    scratch_types=[
        pltpu.VMEM((B_PER_W,), jnp.int32),
        pltpu.VMEM((2 * TAGS, B_PER_W), jnp.int32),
        pltpu.VMEM((ROW_CHUNK, D), jnp.float32),
        pltpu.VMEM((ROW_CHUNK, D), jnp.float32),
        pltpu.VMEM((B_PER_W, D), jnp.float32),
        pltpu.SemaphoreType.DMA,
        pltpu.SemaphoreType.DMA,
        pltpu.SemaphoreType.DMA,
        pltpu.SemaphoreType.DMA,
    ],
)
def _sc_gather_hist(table_hbm, idx_hbm, tags_hbm,
                    rows_hbm, counts_hbm,
                    idx_v, tags_v, rows_a, rows_b, counts_v,
                    tsem, gsem_a, gsem_b, csem):
    wid = lax.axis_index("s") * NC + lax.axis_index("c")
    base = wid * B_PER_W

    tags_dma = pltpu.async_copy(tags_hbm.at[wid], tags_v, tsem)
    pltpu.sync_copy(idx_hbm.at[pl.ds(base, B_PER_W)], idx_v)
    g0 = pltpu.async_copy(
        table_hbm.at[idx_v.at[pl.ds(0, ROW_CHUNK)]], rows_a, gsem_a)
    g1 = pltpu.async_copy(
        table_hbm.at[idx_v.at[pl.ds(ROW_CHUNK, ROW_CHUNK)]], rows_b, gsem_b)

    zeros16 = jnp.zeros((16,), jnp.float32)

    @pl.loop(0, B_PER_W)
    def _(r):
        for u in range(D // 16):
            counts_v[r, pl.ds(u * 16, 16)] = zeros16

    tags_dma.wait()
    ones16 = jnp.ones((16,), jnp.float32)
    iota16 = lax.iota(jnp.int32, 16)

    @pl.loop(0, N_GRP)
    def _(g):
        rows = g * 16 + iota16
        for t in range(2 * TAGS):
            ids16 = tags_v[t, pl.ds(g * 16, 16)]
            plsc.addupdate_scatter(counts_v, [rows, ids16], ones16)

    counts_out = pltpu.async_copy(
        counts_v, counts_hbm.at[pl.ds(base, B_PER_W)], csem)

    g0.wait()
    pltpu.sync_copy(rows_a, rows_hbm.at[pl.ds(base, ROW_CHUNK)])
    g2 = pltpu.async_copy(
        table_hbm.at[idx_v.at[pl.ds(2 * ROW_CHUNK, ROW_CHUNK)]], rows_a,
        gsem_a)
    g1.wait()
    pltpu.sync_copy(rows_b, rows_hbm.at[pl.ds(base + ROW_CHUNK, ROW_CHUNK)])
    g3 = pltpu.async_copy(
        table_hbm.at[idx_v.at[pl.ds(3 * ROW_CHUNK, ROW_CHUNK)]], rows_b,
        gsem_b)
    g2.wait()
    pltpu.sync_copy(rows_a, rows_hbm.at[pl.ds(base + 2 * ROW_CHUNK,
                                              ROW_CHUNK)])
    g3.wait()
    pltpu.sync_copy(rows_b, rows_hbm.at[pl.ds(base + 3 * ROW_CHUNK,
                                              ROW_CHUNK)])
    counts_out.wait()


BLK = 2048


def _mlp_body(genre_ref, counts_ref, tempo_ref, mt_ref, it_ref,
              wt_ref, bt_ref, w1_ref, b1_ref, w2_ref, b2_ref, out_ref):
    tempo = tempo_ref[...]
    tempo_vec = (tempo[:, 0:1] * wt_ref[0:1, :]
                 + tempo[:, 1:2] * wt_ref[1:2, :] + bt_ref[...])

    counts = counts_ref[...]
    mood_vec = jnp.dot(counts[:, :D // 2], mt_ref[...],
                       preferred_element_type=jnp.float32) * (1.0 / TAGS)
    instr_vec = jnp.dot(counts[:, D // 2:], it_ref[...],
                        preferred_element_type=jnp.float32) * (1.0 / TAGS)

    x = jnp.concatenate(
        [genre_ref[...], tempo_vec, mood_vec, instr_vec],
        axis=-1).astype(jnp.bfloat16)
    h = jnp.maximum(
        jnp.dot(x, w1_ref[...], preferred_element_type=jnp.float32)
        + b1_ref[...], 0.0).astype(jnp.bfloat16)
    out_ref[...] = (jnp.dot(h, w2_ref[...], preferred_element_type=jnp.float32)
                    + b2_ref[...])


def _tc_mlp(genre_vec, counts, tempo_range, mood_pad, instr_pad,
            Wt, bt2, W1, b12, W2, b22):
    n_blk = B // BLK
    full = lambda shape: pl.BlockSpec(shape, lambda i: (0, 0))
    return pl.pallas_call(
        _mlp_body,
        grid=(n_blk,),
        in_specs=[
            pl.BlockSpec((BLK, D), lambda i: (i, 0)),
            pl.BlockSpec((BLK, D), lambda i: (i, 0)),
            pl.BlockSpec((BLK, 2), lambda i: (i, 0)),
            full((D // 2, D)),
            full((D // 2, D)),
            full((2, D)),
            full((1, D)),
            full((4 * D, H)),
            full((1, H)),
            full((H, D)),
            full((1, D)),
        ],
        out_specs=pl.BlockSpec((BLK, D), lambda i: (i, 0)),
        out_shape=jax.ShapeDtypeStruct((B, D), jnp.float32),
        compiler_params=pltpu.CompilerParams(
            dimension_semantics=("parallel",)),
    )(genre_vec, counts, tempo_range, mood_pad, instr_pad,
      Wt, bt2, W1, b12, W2, b22)


def kernel(genre_ids, tempo_range, mood_ids, instr_ids, genre_table,
           mood_table, instr_table, Wt, bt, W1, b1, W2, b2):
    ids_comb = jnp.concatenate(
        [mood_ids.astype(jnp.int32), instr_ids.astype(jnp.int32) + D // 2],
        axis=1)
    ids3 = ids_comb.T.reshape(2 * TAGS, NW, B_PER_W).transpose(1, 0, 2)

    genre_vec, counts = _sc_gather_hist(
        genre_table, genre_ids.astype(jnp.int32), ids3)

    pad = jnp.zeros((D // 2 - N_SMALL, D), jnp.float32)
    mood_pad = jnp.concatenate([mood_table, pad], axis=0)
    instr_pad = jnp.concatenate([instr_table, pad], axis=0)

    return _tc_mlp(genre_vec, counts, tempo_range, mood_pad, instr_pad,
                   Wt, bt.reshape(1, D), W1.astype(jnp.bfloat16),
                   b1.reshape(1, H), W2.astype(jnp.bfloat16),
                   b2.reshape(1, D))


# speedup vs baseline: 1.6265x; 1.1222x over previous
import dataclasses
import functools

import jax
import jax.numpy as jnp
from jax import lax
from jax.experimental import pallas as pl
from jax.experimental.pallas import tpu as pltpu
from jax.experimental.pallas import tpu_sc as plsc

B = 16384
D = 128
TAGS = 20
N_SMALL = 50
H = 256

NC = 2
NS = 16
NW = NC * NS
B_PER_W = B // NW
ROW_CHUNK = B_PER_W // 4
N_GRP = B_PER_W // 16

_sc_mesh = plsc.VectorSubcoreMesh(core_axis_name="c", subcore_axis_name="s")

_sc_cp = pltpu.CompilerParams()
if "needs_layout_passes" in pltpu.CompilerParams.__dataclass_fields__:
    _sc_cp = dataclasses.replace(_sc_cp, needs_layout_passes=False)


@functools.partial(
    pl.kernel,
    mesh=_sc_mesh,
    compiler_params=_sc_cp,
    out_type=(
        jax.ShapeDtypeStruct((B, D), jnp.float32),
        jax.ShapeDtypeStruct((B, D), jnp.float32),
    ),
    scratch_types=[
        pltpu.VMEM((B_PER_W,), jnp.int32),
        pltpu.VMEM((2 * TAGS, B_PER_W), jnp.int32),
        pltpu.VMEM((ROW_CHUNK, D), jnp.float32),
        pltpu.VMEM((ROW_CHUNK, D), jnp.float32),
        pltpu.VMEM((B_PER_W, D), jnp.float32),
        pltpu.SemaphoreType.DMA,
        pltpu.SemaphoreType.DMA,
        pltpu.SemaphoreType.DMA,
        pltpu.SemaphoreType.DMA,
    ],
)
def _sc_gather_hist(table_hbm, idx_hbm, tags_hbm,
                    rows_hbm, counts_hbm,
                    idx_v, tags_v, rows_a, rows_b, counts_v,
                    tsem, gsem_a, gsem_b, csem):
    wid = lax.axis_index("s") * NC + lax.axis_index("c")
    base = wid * B_PER_W

    tags_dma = pltpu.async_copy(tags_hbm.at[wid], tags_v, tsem)
    pltpu.sync_copy(idx_hbm.at[pl.ds(base, B_PER_W)], idx_v)
    g0 = pltpu.async_copy(
        table_hbm.at[idx_v.at[pl.ds(0, ROW_CHUNK)]], rows_a, gsem_a)
    g1 = pltpu.async_copy(
        table_hbm.at[idx_v.at[pl.ds(ROW_CHUNK, ROW_CHUNK)]], rows_b, gsem_b)

    zeros16 = jnp.zeros((16,), jnp.float32)

    @plsc.parallel_loop(0, B_PER_W, unroll=2)
    def _(r):
        for u in range(D // 16):
            counts_v[r, pl.ds(u * 16, 16)] = zeros16

    tags_dma.wait()
    ones16 = jnp.ones((16,), jnp.float32)
    iota16 = lax.iota(jnp.int32, 16)

    @plsc.parallel_loop(0, N_GRP, unroll=2)
    def _(g):
        rows = g * 16 + iota16
        for t in range(2 * TAGS):
            ids16 = tags_v[t, pl.ds(g * 16, 16)]
            plsc.addupdate_scatter(counts_v, [rows, ids16], ones16)

    counts_out = pltpu.async_copy(
        counts_v, counts_hbm.at[pl.ds(base, B_PER_W)], csem)

    g0.wait()
    pltpu.sync_copy(rows_a, rows_hbm.at[pl.ds(base, ROW_CHUNK)])
    g2 = pltpu.async_copy(
        table_hbm.at[idx_v.at[pl.ds(2 * ROW_CHUNK, ROW_CHUNK)]], rows_a,
        gsem_a)
    g1.wait()
    pltpu.sync_copy(rows_b, rows_hbm.at[pl.ds(base + ROW_CHUNK, ROW_CHUNK)])
    g3 = pltpu.async_copy(
        table_hbm.at[idx_v.at[pl.ds(3 * ROW_CHUNK, ROW_CHUNK)]], rows_b,
        gsem_b)
    g2.wait()
    pltpu.sync_copy(rows_a, rows_hbm.at[pl.ds(base + 2 * ROW_CHUNK,
                                              ROW_CHUNK)])
    g3.wait()
    pltpu.sync_copy(rows_b, rows_hbm.at[pl.ds(base + 3 * ROW_CHUNK,
                                              ROW_CHUNK)])
    counts_out.wait()


BLK = 2048


def _mlp_body(genre_ref, counts_ref, tempo_ref, mt_ref, it_ref,
              wt_ref, bt_ref, w1_ref, b1_ref, w2_ref, b2_ref, out_ref):
    tempo = tempo_ref[...]
    tempo_vec = (tempo[:, 0:1] * wt_ref[0:1, :]
                 + tempo[:, 1:2] * wt_ref[1:2, :] + bt_ref[...])

    counts = counts_ref[...].astype(jnp.bfloat16)
    mood_vec = jnp.dot(counts[:, :D // 2], mt_ref[...],
                       preferred_element_type=jnp.float32) * (1.0 / TAGS)
    instr_vec = jnp.dot(counts[:, D // 2:], it_ref[...],
                        preferred_element_type=jnp.float32) * (1.0 / TAGS)

    x = jnp.concatenate(
        [genre_ref[...], tempo_vec, mood_vec, instr_vec],
        axis=-1).astype(jnp.bfloat16)
    h = jnp.maximum(
        jnp.dot(x, w1_ref[...], preferred_element_type=jnp.float32)
        + b1_ref[...], 0.0).astype(jnp.bfloat16)
    out_ref[...] = (jnp.dot(h, w2_ref[...], preferred_element_type=jnp.float32)
                    + b2_ref[...])


def _tc_mlp(genre_vec, counts, tempo_range, mood_pad, instr_pad,
            Wt, bt2, W1, b12, W2, b22):
    n_blk = B // BLK
    full = lambda shape: pl.BlockSpec(shape, lambda i: (0, 0))
    return pl.pallas_call(
        _mlp_body,
        grid=(n_blk,),
        in_specs=[
            pl.BlockSpec((BLK, D), lambda i: (i, 0)),
            pl.BlockSpec((BLK, D), lambda i: (i, 0)),
            pl.BlockSpec((BLK, 2), lambda i: (i, 0)),
            full((D // 2, D)),
            full((D // 2, D)),
            full((2, D)),
            full((1, D)),
            full((4 * D, H)),
            full((1, H)),
            full((H, D)),
            full((1, D)),
        ],
        out_specs=pl.BlockSpec((BLK, D), lambda i: (i, 0)),
        out_shape=jax.ShapeDtypeStruct((B, D), jnp.float32),
        compiler_params=pltpu.CompilerParams(
            dimension_semantics=("parallel",)),
    )(genre_vec, counts, tempo_range, mood_pad, instr_pad,
      Wt, bt2, W1, b12, W2, b22)


def kernel(genre_ids, tempo_range, mood_ids, instr_ids, genre_table,
           mood_table, instr_table, Wt, bt, W1, b1, W2, b2):
    ids_comb = jnp.concatenate(
        [mood_ids.astype(jnp.int32), instr_ids.astype(jnp.int32) + D // 2],
        axis=1)
    ids3 = ids_comb.T.reshape(2 * TAGS, NW, B_PER_W).transpose(1, 0, 2)

    genre_vec, counts = _sc_gather_hist(
        genre_table, genre_ids.astype(jnp.int32), ids3)

    pad = jnp.zeros((D // 2 - N_SMALL, D), jnp.bfloat16)
    mood_pad = jnp.concatenate([mood_table.astype(jnp.bfloat16), pad], axis=0)
    instr_pad = jnp.concatenate([instr_table.astype(jnp.bfloat16), pad],
                                axis=0)

    return _tc_mlp(genre_vec, counts, tempo_range, mood_pad, instr_pad,
                   Wt, bt.reshape(1, D), W1.astype(jnp.bfloat16),
                   b1.reshape(1, H), W2.astype(jnp.bfloat16),
                   b2.reshape(1, D))
